# baseline (speedup 1.0000x reference)
import functools

import jax
import jax.numpy as jnp
from jax import lax
from jax.experimental import pallas as pl
from jax.experimental.pallas import tpu as pltpu
from jax.experimental.pallas import tpu_sc as plsc

N, D, B = 1000000, 64, 16384
C = 128
NC, NS = 2, 16
NW = NC * NS
CPW = B // C // NW


def _body(inp, mem, sp, dp, dn, ds, dn0, o1, o2, o3, o4, o5,
          sp_v, dp_v, dn_v, ds_v, dn0_v, row, gsem, ssem):
    wid = lax.axis_index("s") * NC + lax.axis_index("c")
    cbase = wid * CPW
    cp = pltpu.sync_copy
    cp(sp.at[pl.ds(cbase, CPW)], sp_v)
    cp(dp.at[pl.ds(cbase, CPW)], dp_v)
    cp(dn.at[pl.ds(cbase, CPW)], dn_v)
    cp(ds.at[pl.ds(cbase, CPW)], ds_v)
    cp(dn0.at[pl.ds(cbase, CPW)], dn0_v)

    def task(tbl, gidx, out, sidx):
        def chunk(j, carry):
            pltpu.async_copy(tbl.at[gidx.at[j]], row, gsem).wait()
            if sidx is None:
                cp(row, out.at[pl.ds((cbase + j) * C, C)])
            else:
                pltpu.async_copy(row, out.at[sidx.at[j]], ssem).wait()
            return carry
        lax.fori_loop(0, CPW, chunk, 0)

    task(inp, sp_v, o1, None)
    task(inp, dp_v, o2, ds_v)
    task(inp, dn_v, o3, dn0_v)
    task(mem, dp_v, o4, ds_v)
    task(mem, sp_v, o5, ds_v)


_out = jax.ShapeDtypeStruct((B, D), jnp.float32)
_sc_call = functools.partial(
    pl.kernel,
    out_type=(_out,) * 5,
    mesh=plsc.VectorSubcoreMesh(core_axis_name="c", subcore_axis_name="s"),
    scratch_types=[
        pltpu.VMEM((CPW, C), jnp.int32),
        pltpu.VMEM((CPW, C), jnp.int32),
        pltpu.VMEM((CPW, C), jnp.int32),
        pltpu.VMEM((CPW, C), jnp.int32),
        pltpu.VMEM((CPW, C), jnp.int32),
        pltpu.VMEM((C, D), jnp.float32),
        pltpu.SemaphoreType.DMA,
        pltpu.SemaphoreType.DMA,
    ],
    compiler_params=pltpu.CompilerParams(use_tc_tiling_on_sc=False),
)(_body)


def kernel(input, memory, src_pos_index, dst_pos_index, dst_neg_index,
           dist_src_index, dist_neg_src_index, neg_samples):
    del neg_samples
    sp = src_pos_index.reshape(B // C, C)
    dp = dst_pos_index.reshape(B // C, C)
    dn = dst_neg_index.reshape(B // C, C)
    ds = dist_src_index.reshape(B // C, C)
    dn0 = (dist_neg_src_index - B).reshape(B // C, C)
    return _sc_call(input, memory, sp, dp, dn, ds, dn0)


# speedup vs baseline: 1.0101x; 1.0101x over previous
import functools

import jax
import jax.numpy as jnp
from jax import lax
from jax.experimental import pallas as pl
from jax.experimental.pallas import tpu as pltpu
from jax.experimental.pallas import tpu_sc as plsc

N, D, B = 1000000, 64, 16384
NC, NS = 2, 16
NW = NC * NS
RPW = B // NW


def _body(inp, mem, sp, dp, dn, ds, dn0, idn, o1, o2, o3, o4, o5,
          sp_v, dp_v, dn_v, ds_v, dn0_v, id_v, rb_a, rb_b,
          gs_a, gs_b, ss_a, ss_b):
    wid = lax.axis_index("s") * NC + lax.axis_index("c")
    base = wid * RPW
    cp = pltpu.sync_copy
    cp(sp.at[pl.ds(base, RPW)], sp_v)
    cp(dp.at[pl.ds(base, RPW)], dp_v)
    cp(dn.at[pl.ds(base, RPW)], dn_v)
    cp(ds.at[pl.ds(base, RPW)], ds_v)
    cp(dn0.at[pl.ds(base, RPW)], dn0_v)
    cp(idn.at[pl.ds(base, RPW)], id_v)

    tasks = [
        (inp, sp_v, o1, id_v),
        (inp, dp_v, o2, ds_v),
        (inp, dn_v, o3, dn0_v),
        (mem, dp_v, o4, ds_v),
        (mem, sp_v, o5, ds_v),
    ]
    rbufs = (rb_a, rb_b)
    gsems = (gs_a, gs_b)
    ssems = (ss_a, ss_b)

    def gather(t):
        tbl, gidx, _, _ = tasks[t]
        pltpu.async_copy(tbl.at[gidx], rbufs[t % 2], gsems[t % 2])

    def scatter_start(t):
        _, _, out, sidx = tasks[t]
        return pltpu.async_copy(rbufs[t % 2], out.at[sidx], ssems[t % 2])

    gather(0)
    pend = [None, None]
    for t in range(5):
        if t + 1 < 5:
            if pend[(t + 1) % 2] is not None:
                pend[(t + 1) % 2].wait()
            gather(t + 1)
        pltpu.make_async_copy(tasks[t][0].at[tasks[t][1]],
                              rbufs[t % 2], gsems[t % 2]).wait()
        pend[t % 2] = scatter_start(t)
    pend[0].wait()
    pend[1].wait()


_out = jax.ShapeDtypeStruct((B, D), jnp.float32)
_sc_call = functools.partial(
    pl.kernel,
    out_type=(_out,) * 5,
    mesh=plsc.VectorSubcoreMesh(core_axis_name="c", subcore_axis_name="s"),
    scratch_types=[
        pltpu.VMEM((RPW,), jnp.int32),
        pltpu.VMEM((RPW,), jnp.int32),
        pltpu.VMEM((RPW,), jnp.int32),
        pltpu.VMEM((RPW,), jnp.int32),
        pltpu.VMEM((RPW,), jnp.int32),
        pltpu.VMEM((RPW,), jnp.int32),
        pltpu.VMEM((RPW, D), jnp.float32),
        pltpu.VMEM((RPW, D), jnp.float32),
        pltpu.SemaphoreType.DMA,
        pltpu.SemaphoreType.DMA,
        pltpu.SemaphoreType.DMA,
        pltpu.SemaphoreType.DMA,
    ],
    compiler_params=pltpu.CompilerParams(use_tc_tiling_on_sc=False),
)(_body)


def kernel(input, memory, src_pos_index, dst_pos_index, dst_neg_index,
           dist_src_index, dist_neg_src_index, neg_samples):
    del neg_samples
    dn0 = dist_neg_src_index - B
    idn = jnp.arange(B, dtype=jnp.int32)
    return _sc_call(input, memory, src_pos_index, dst_pos_index,
                    dst_neg_index, dist_src_index, dn0, idn)


# speedup vs baseline: 2.1843x; 2.1624x over previous
import functools

import jax
import jax.numpy as jnp
from jax import lax
from jax.experimental import pallas as pl
from jax.experimental.pallas import tpu as pltpu
from jax.experimental.pallas import tpu_sc as plsc

N, D, B = 1000000, 64, 16384
NT = N // 8
NC, NS = 2, 16
NW = NC * NS
RPW = B // NW
K = 64
NCHUNK = RPW // K
G = K // 16


def _body(inp3, mem3, sp, dp, dn, ds, dn0, o1, o2, o3, o4, o5,
          sp_v, dp_v, dn_v, ds_v, dn0_v, stg_a, stg_b,
          gs_a, gs_b, ss_a, ss_b):
    wid = lax.axis_index("s") * NC + lax.axis_index("c")
    base = wid * RPW
    cbase = wid * NCHUNK
    cp = pltpu.sync_copy
    cp(sp.at[pl.ds(cbase, NCHUNK)], sp_v)
    cp(dp.at[pl.ds(cbase, NCHUNK)], dp_v)
    cp(dn.at[pl.ds(cbase, NCHUNK)], dn_v)
    cp(ds.at[pl.ds(cbase, NCHUNK)], ds_v)
    cp(dn0.at[pl.ds(cbase, NCHUNK)], dn0_v)

    def fire_gathers(tbl, gidx_v, c, stg, gsem):
        descs = []
        for g in range(G):
            r16 = gidx_v[c, pl.ds(g * 16, 16)]
            t16 = lax.shift_right_logical(r16, 3)
            s16 = lax.bitwise_and(r16, 7)
            for u in range(16):
                descs.append(pltpu.async_copy(
                    inp3.at[t16[u], s16[u]] if tbl is None
                    else tbl.at[t16[u], s16[u]],
                    stg.at[g * 16 + u, pl.ds(0, D)], gsem))
        return descs

    def fire_scatter(out, sidx_v, c, stg, ssem):
        if sidx_v is None:
            return pltpu.async_copy(
                stg, out.at[pl.ds(base + c * K, K)], ssem)
        return pltpu.async_copy(stg, out.at[sidx_v.at[c]], ssem)

    def task(tbl, gidx_v, out, sidx_v):
        def pair(k, carry):
            c0 = 2 * k
            c1 = 2 * k + 1
            ga = fire_gathers(tbl, gidx_v, c0, stg_a, gs_a)
            gb = fire_gathers(tbl, gidx_v, c1, stg_b, gs_b)
            for d in ga:
                d.wait()
            sa = fire_scatter(out, sidx_v, c0, stg_a, ss_a)
            for d in gb:
                d.wait()
            sb = fire_scatter(out, sidx_v, c1, stg_b, ss_b)
            sa.wait()
            sb.wait()
            return carry
        lax.fori_loop(0, NCHUNK // 2, pair, 0)

    task(inp3, sp_v, o1, None)
    task(inp3, dp_v, o2, ds_v)
    task(inp3, dn_v, o3, dn0_v)
    task(mem3, dp_v, o4, ds_v)
    task(mem3, sp_v, o5, ds_v)


_out = jax.ShapeDtypeStruct((B, 128), jnp.float32)
_sc_call = functools.partial(
    pl.kernel,
    out_type=(_out,) * 5,
    mesh=plsc.VectorSubcoreMesh(core_axis_name="c", subcore_axis_name="s"),
    scratch_types=[
        pltpu.VMEM((NCHUNK, K), jnp.int32),
        pltpu.VMEM((NCHUNK, K), jnp.int32),
        pltpu.VMEM((NCHUNK, K), jnp.int32),
        pltpu.VMEM((NCHUNK, K), jnp.int32),
        pltpu.VMEM((NCHUNK, K), jnp.int32),
        pltpu.VMEM((K, 128), jnp.float32),
        pltpu.VMEM((K, 128), jnp.float32),
        pltpu.SemaphoreType.DMA,
        pltpu.SemaphoreType.DMA,
        pltpu.SemaphoreType.DMA,
        pltpu.SemaphoreType.DMA,
    ],
    compiler_params=pltpu.CompilerParams(use_tc_tiling_on_sc=True,
                                         needs_layout_passes=False),
)(_body)


def kernel(input, memory, src_pos_index, dst_pos_index, dst_neg_index,
           dist_src_index, dist_neg_src_index, neg_samples):
    del neg_samples
    inp3 = input.reshape(NT, 8, D)
    mem3 = memory.reshape(NT, 8, D)
    sp = src_pos_index.reshape(B // K, K)
    dp = dst_pos_index.reshape(B // K, K)
    dn = dst_neg_index.reshape(B // K, K)
    ds = dist_src_index.reshape(B // K, K)
    dn0 = (dist_neg_src_index - B).reshape(B // K, K)
    o1, o2, o3, o4, o5 = _sc_call(inp3, mem3, sp, dp, dn, ds, dn0)
    return (o1[:, :D], o2[:, :D], o3[:, :D], o4[:, :D], o5[:, :D])
